# baseline (speedup 1.0000x reference)
import functools

import jax
import jax.numpy as jnp
from jax import lax
from jax.experimental import pallas as pl
from jax.experimental.pallas import tpu as pltpu
from jax.experimental.pallas import tpu_sc as plsc

_LANE_TILE = 128
_VREG = 16


def _table_body(bulk, x_ref, p_ref, w_ref, bias_ref, oa_ref, ob_ref):
    x = x_ref[...]
    p = p_ref[...]
    inv_d = 1.0 / x.shape[-1]
    for i in range(3):
        g = p[2 * i : 2 * i + 1, :]
        b = p[2 * i + 1 : 2 * i + 2, :]
        m = jnp.sum(x, axis=-1, keepdims=True) * inv_d
        ms = jnp.sum(x * x, axis=-1, keepdims=True) * inv_d
        x = (x - m) * lax.rsqrt(ms - m * m + 1e-5) * g + b
    y = jnp.dot(x, w_ref[...], preferred_element_type=jnp.float32) + bias_ref[...]
    n = y.shape[-1]
    oa_ref[...] = y[:, :bulk]
    ob_ref[...] = y[:, n - _LANE_TILE :]


@functools.cache
def _table_head(vocab, d, vocab_out):
    bulk = (vocab_out // _LANE_TILE) * _LANE_TILE
    return pl.pallas_call(
        functools.partial(_table_body, bulk),
        grid=(1,),
        in_specs=[
            pl.BlockSpec((vocab, d), lambda i: (0, 0)),
            pl.BlockSpec((6, d), lambda i: (0, 0)),
            pl.BlockSpec((d, vocab_out), lambda i: (0, 0)),
            pl.BlockSpec((1, vocab_out), lambda i: (0, 0)),
        ],
        out_specs=[
            pl.BlockSpec((vocab, bulk), lambda i: (0, 0)),
            pl.BlockSpec((vocab, _LANE_TILE), lambda i: (0, 0)),
        ],
        out_shape=[
            jax.ShapeDtypeStruct((vocab, bulk), jnp.float32),
            jax.ShapeDtypeStruct((vocab, _LANE_TILE), jnp.float32),
        ],
    )


@functools.cache
def _sc_expand(vocab, vocab_out, rows, cols):
    bulk = (vocab_out // _LANE_TILE) * _LANE_TILE
    toff = vocab_out - _LANE_TILE
    info = plsc.get_sparse_core_info()
    nw = info.num_cores * info.num_subcores
    assert rows % (2 * nw) == 0
    r_per_w = rows // nw

    mesh = plsc.VectorSubcoreMesh(core_axis_name="c", subcore_axis_name="s")

    @functools.partial(
        pl.kernel,
        mesh=mesh,
        out_type=jax.ShapeDtypeStruct((rows, cols, vocab_out), jnp.float32),
        scratch_types=[
            pltpu.VMEM((r_per_w, cols), jnp.int32),
            pltpu.VMEM((cols, vocab_out), jnp.float32),
            pltpu.VMEM((cols, vocab_out), jnp.float32),
            pltpu.VMEM((cols, _LANE_TILE), jnp.float32),
            pltpu.SemaphoreType.DMA,
            pltpu.SemaphoreType.DMA,
        ],
    )
    def expand(ta_hbm, tb_hbm, idx_hbm, out_hbm, idx_v, b0, b1, c, s0, s1):
        wid = lax.axis_index("s") * info.num_cores + lax.axis_index("c")
        base = wid * r_per_w
        pltpu.sync_copy(idx_hbm.at[pl.ds(base, r_per_w), :], idx_v)
        bufs = ((b0, s0), (b1, s1))

        def fire(j, slot):
            b, s = bufs[slot]
            ia = pltpu.async_copy(ta_hbm.at[idx_v.at[j, :]], b.at[:, pl.ds(0, bulk)], s)
            ib = pltpu.async_copy(tb_hbm.at[idx_v.at[j, :]], c, s)
            return ia, ib

        def fold_tail(slot):
            b, _ = bufs[slot]
            for r in range(cols):
                for k in range(_LANE_TILE // _VREG):
                    b[r, pl.ds(toff + k * _VREG, _VREG)] = c[r, pl.ds(k * _VREG, _VREG)]

        def body(t, _):
            j0 = 2 * t
            h0 = fire(j0, 0)
            for h in h0:
                h.wait()
            fold_tail(0)
            h1 = fire(j0 + 1, 1)
            pltpu.sync_copy(b0, out_hbm.at[base + j0])
            for h in h1:
                h.wait()
            fold_tail(1)
            pltpu.sync_copy(b1, out_hbm.at[base + j0 + 1])
            return _

        lax.fori_loop(0, r_per_w // 2, body, 0)

    return expand


def kernel(indices, emb, g0, b0, g1, b1, gf, bf, W, bias):
    vocab, d = emb.shape
    vocab_out = W.shape[1]
    rows, cols = indices.shape
    params = jnp.stack([g0, b0, g1, b1, gf, bf], axis=0)

    ta, tb = _table_head(vocab, d, vocab_out)(
        emb, params, W, bias.reshape(1, vocab_out)
    )
    out = _sc_expand(vocab, vocab_out, rows, cols)(ta, tb, indices.astype(jnp.int32))
    return out
